# Initial kernel scaffold; baseline (speedup 1.0000x reference)
import jax
import jax.numpy as jnp
from jax.experimental import pallas as pl


def kernel(query, reference_points, input_flatten, input_spatial_shapes, add_keys, input_level_start_index, sampling_offsets_W, sampling_offsets_b, attn_W, value_W, value_b, head_mixer_W):
    raise NotImplementedError("write your pallas kernel here")


# speedup vs baseline: 6.4715x; 6.4715x over previous
import functools

import jax
import jax.numpy as jnp
from jax import lax
from jax.experimental import pallas as pl
from jax.experimental.pallas import tpu as pltpu
from jax.experimental.pallas import tpu_sc as plsc

H, L, P, NQ, D = 8, 4, 4, 1024, 256
SLEV = (128.0, 64.0, 32.0, 16.0)
START = (0, 16384, 20480, 21504)
NROWS = 32 * 4096
SC_CH = 128
SC_NCH = 32


def _idx_body(q2_ref, rp_ref, w_ref, b_ref, out_ref):
    ot = lax.dot_general(w_ref[...], q2_ref[...], (((1,), (1,)), ((), ())),
                         preferred_element_type=jnp.float32) + b_ref[...]
    ot3 = ot.reshape(128, 2, NQ)
    for g in range(32):
        lvl = g % 4
        s = SLEV[lvl]
        xg = ot3[4 * g:4 * g + 4, 0, :]
        yg = ot3[4 * g:4 * g + 4, 1, :]
        lx = jnp.clip(rp_ref[lvl, 0:1, :] + xg * (1.0 / s), 0.0, 0.999)
        ly = jnp.clip(rp_ref[lvl, 1:2, :] + yg * (1.0 / s), 0.0, 0.999)
        ix = (lx * s).astype(jnp.int32)
        iy = (ly * s).astype(jnp.int32)
        out_ref[g] = ix + iy * int(s) + START[lvl]


def _sc_gather_body(table_hbm, idx_hbm, out_hbm, idx_v, rows_v, sem):
    w = lax.axis_index("s") * 2 + lax.axis_index("c")
    pltpu.sync_copy(idx_hbm.at[w], idx_v)

    def body(c, carry):
        pltpu.async_copy(table_hbm.at[idx_v.at[c]], rows_v, sem).wait()
        pltpu.sync_copy(rows_v, out_hbm.at[pl.ds(w * 4096 + c * SC_CH, SC_CH)])
        return carry

    lax.fori_loop(0, SC_NCH, body, 0)


def _head_body(g_ref, q2_ref, qp_ref, aw_ref, awx_ref, vw0_ref, vw1_ref,
               vb0_ref, vb1_ref, ak_ref, hm_ref, out_ref):
    h = pl.program_id(0)
    q2 = q2_ref[...]
    slot_cols = []
    for lvl in range(L):
        simil = aw_ref[lvl]
        drs = []
        for r in range(16):
            kr = g_ref[lvl, 256 * r:256 * (r + 1), :]
            dr = lax.dot_general(qp_ref[r], kr, (((1,), (0,)), ((), ())),
                                 preferred_element_type=jnp.float32)
            drs.append(dr.reshape(64, 1, 256))
        d2 = jnp.concatenate(drs, axis=1).reshape(NQ, 256)
        t = lax.dot_general(d2, simil, (((1,), (1,)), ((), ())),
                            preferred_element_type=jnp.float32)
        t3 = t.reshape(NQ, 4, 64)
        c_iota = lax.broadcasted_iota(jnp.int32, (NQ, 4, 64), 2)
        q_iota = lax.broadcasted_iota(jnp.int32, (NQ, 4, 64), 0)
        sel = c_iota == (q_iota // 16)
        slot_cols.append(jnp.sum(jnp.where(sel, t3, 0.0), axis=2))
    ak = ak_ref[...]
    ki2 = lax.dot_general(ak, awx_ref[0], (((1,), (1,)), ((), ())),
                          preferred_element_type=jnp.float32)
    attn_add = lax.dot_general(q2, ki2, (((1,), (1,)), ((), ())),
                               preferred_element_type=jnp.float32)
    logits = jnp.concatenate(slot_cols + [attn_add], axis=1)
    m = jnp.max(logits, axis=1, keepdims=True)
    e = jnp.exp(logits - m)
    a = e / jnp.sum(e, axis=1, keepdims=True)
    wsum = jnp.zeros((NQ, 256), jnp.float32)
    for lvl in range(L):
        for p in range(4):
            wsum = wsum + a[:, lvl * 4 + p:lvl * 4 + p + 1] * \
                g_ref[lvl, 1024 * p:1024 * (p + 1), :]
    a16 = jnp.sum(a[:, :16], axis=1, keepdims=True)
    v_main = lax.dot_general(wsum, vw0_ref[0], (((1,), (1,)), ((), ())),
                             preferred_element_type=jnp.float32) + a16 * vb0_ref[0]
    v2 = lax.dot_general(ak, vw1_ref[0], (((1,), (1,)), ((), ())),
                         preferred_element_type=jnp.float32) + vb1_ref[0]
    v_add = lax.dot_general(a[:, 16:], v2, (((1,), (0,)), ((), ())),
                            preferred_element_type=jnp.float32)
    hm = hm_ref[...]
    hme = jnp.exp(hm - jnp.max(hm, axis=0, keepdims=True))
    hw = hme / jnp.sum(hme, axis=0, keepdims=True)
    hsel = lax.broadcasted_iota(jnp.int32, (9, 1), 0) == h
    hwh = jnp.sum(jnp.where(hsel, hw, 0.0), axis=0, keepdims=True)
    contrib = (v_main + v_add) * hwh

    @pl.when(h == 0)
    def _():
        out_ref[...] = q2 * hw[8:9, :] + contrib

    @pl.when(h != 0)
    def _():
        out_ref[...] = out_ref[...] + contrib


def _compute_idx(q2, rp_t, w, b2d):
    return pl.pallas_call(
        _idx_body,
        out_shape=jax.ShapeDtypeStruct((32, 4, NQ), jnp.int32),
    )(q2, rp_t, w, b2d)


@functools.lru_cache(maxsize=1)
def _sc_gather_fn():
    return functools.partial(
        pl.kernel,
        mesh=plsc.VectorSubcoreMesh(core_axis_name="c", subcore_axis_name="s"),
        out_type=jax.ShapeDtypeStruct((NROWS, D), jnp.float32),
        scratch_types=[
            pltpu.VMEM((SC_NCH, SC_CH), jnp.int32),
            pltpu.VMEM((SC_CH, D), jnp.float32),
            pltpu.SemaphoreType.DMA,
        ],
    )(_sc_gather_body)


def _head_stage(g, q2, qp, aw_main, aw_extra, vw0, vw1, vb0, vb1, ak, hm_t):
    return pl.pallas_call(
        _head_body,
        grid=(H,),
        in_specs=[
            pl.BlockSpec((4, 4096, 256), lambda h: (h, 0, 0)),
            pl.BlockSpec((NQ, 256), lambda h: (0, 0)),
            pl.BlockSpec((16, 64, 256), lambda h: (0, 0, 0)),
            pl.BlockSpec((4, 256, 256), lambda h: (h, 0, 0)),
            pl.BlockSpec((1, 256, 256), lambda h: (h, 0, 0)),
            pl.BlockSpec((1, 256, 256), lambda h: (h, 0, 0)),
            pl.BlockSpec((1, 256, 256), lambda h: (h, 0, 0)),
            pl.BlockSpec((1, 1, 256), lambda h: (h, 0, 0)),
            pl.BlockSpec((1, 1, 256), lambda h: (h, 0, 0)),
            pl.BlockSpec((64, 256), lambda h: (0, 0)),
            pl.BlockSpec((9, 256), lambda h: (0, 0)),
        ],
        out_specs=pl.BlockSpec((NQ, 256), lambda h: (0, 0)),
        out_shape=jax.ShapeDtypeStruct((NQ, 256), jnp.float32),
    )(g, q2, qp, aw_main, aw_extra, vw0, vw1, vb0, vb1, ak, hm_t)


def kernel(query, reference_points, input_flatten, input_spatial_shapes,
           add_keys, input_level_start_index, sampling_offsets_W,
           sampling_offsets_b, attn_W, value_W, value_b, head_mixer_W):
    q2 = query[0]
    rp_t = reference_points[0].transpose(1, 2, 0)
    b2d = sampling_offsets_b.reshape(256, 1)
    idx = _compute_idx(q2, rp_t, sampling_offsets_W, b2d)
    idx3 = idx.reshape(32, SC_NCH, SC_CH)
    g = _sc_gather_fn()(input_flatten[0], idx3)
    g3 = g.reshape(32, 4096, 256)
    qp = q2.reshape(64, 16, 256).transpose(1, 0, 2)
    out = _head_stage(
        g3, q2, qp,
        attn_W[:32], attn_W[4::4][:8],
        value_W[0::2], value_W[1::2],
        value_b[0::2].reshape(8, 1, 256), value_b[1::2].reshape(8, 1, 256),
        add_keys[0], head_mixer_W.T,
    )
    return out[None]


# speedup vs baseline: 7.4869x; 1.1569x over previous
import functools

import jax
import jax.numpy as jnp
from jax import lax
from jax.experimental import pallas as pl
from jax.experimental.pallas import tpu as pltpu
from jax.experimental.pallas import tpu_sc as plsc

H, L, P, NQ, D = 8, 4, 4, 1024, 256
SLEV = (128.0, 64.0, 32.0, 16.0)
START = (0, 16384, 20480, 21504)
NROWS = 32 * 4096
SC_CH = 128
SC_NCH = 32


def _idx_body(q2_ref, rp_ref, w_ref, b_ref, out_ref):
    ot = lax.dot_general(w_ref[...], q2_ref[...], (((1,), (1,)), ((), ())),
                         preferred_element_type=jnp.float32) + b_ref[...]
    ot3 = ot.reshape(128, 2, NQ)
    for g in range(32):
        lvl = g % 4
        s = SLEV[lvl]
        xg = ot3[4 * g:4 * g + 4, 0, :]
        yg = ot3[4 * g:4 * g + 4, 1, :]
        lx = jnp.clip(rp_ref[lvl, 0:1, :] + xg * (1.0 / s), 0.0, 0.999)
        ly = jnp.clip(rp_ref[lvl, 1:2, :] + yg * (1.0 / s), 0.0, 0.999)
        ix = (lx * s).astype(jnp.int32)
        iy = (ly * s).astype(jnp.int32)
        out_ref[g] = ix + iy * int(s) + START[lvl]


def _sc_gather_body(table_hbm, idx_hbm, out_hbm, idx_v, buf0, buf1,
                    si0, si1, so0, so1):
    w = lax.axis_index("s") * 2 + lax.axis_index("c")
    pltpu.sync_copy(idx_hbm.at[w], idx_v)
    base = w * 4096
    pltpu.async_copy(table_hbm.at[idx_v.at[0]], buf0, si0)
    pltpu.async_copy(table_hbm.at[idx_v.at[1]], buf1, si1)

    def pair(t, carry):
        c0 = 2 * t
        pltpu.make_async_copy(table_hbm.at[idx_v.at[c0]], buf0, si0).wait()
        pltpu.async_copy(buf0, out_hbm.at[pl.ds(base + c0 * SC_CH, SC_CH)], so0)
        pltpu.make_async_copy(table_hbm.at[idx_v.at[c0 + 1]], buf1, si1).wait()
        pltpu.async_copy(
            buf1, out_hbm.at[pl.ds(base + (c0 + 1) * SC_CH, SC_CH)], so1)

        @pl.when(t < SC_NCH // 2 - 1)
        def _():
            pltpu.make_async_copy(
                buf0, out_hbm.at[pl.ds(base + c0 * SC_CH, SC_CH)], so0).wait()
            pltpu.async_copy(table_hbm.at[idx_v.at[c0 + 2]], buf0, si0)
            pltpu.make_async_copy(
                buf1, out_hbm.at[pl.ds(base + (c0 + 1) * SC_CH, SC_CH)],
                so1).wait()
            pltpu.async_copy(table_hbm.at[idx_v.at[c0 + 3]], buf1, si1)

        return carry

    lax.fori_loop(0, SC_NCH // 2, pair, 0)
    last = SC_NCH - 2
    pltpu.make_async_copy(
        buf0, out_hbm.at[pl.ds(base + last * SC_CH, SC_CH)], so0).wait()
    pltpu.make_async_copy(
        buf1, out_hbm.at[pl.ds(base + (last + 1) * SC_CH, SC_CH)], so1).wait()


def _head_body(g_ref, q2_ref, qp_ref, aw_ref, awx_ref, vw0_ref, vw1_ref,
               vb0_ref, vb1_ref, ak_ref, hm_ref, out_ref):
    h = pl.program_id(0)
    q2 = q2_ref[...]
    lane_iota = lax.broadcasted_iota(jnp.int32, (NQ, 256), 1)
    q16 = lax.broadcasted_iota(jnp.int32, (NQ, 256), 0) // 16
    slot_cols = []
    for lvl in range(L):
        simil = aw_ref[lvl]
        drs = []
        for r in range(16):
            kr = g_ref[lvl, 256 * r:256 * (r + 1), :]
            dr = lax.dot_general(qp_ref[r], kr, (((1,), (0,)), ((), ())),
                                 preferred_element_type=jnp.float32)
            drs.append(dr.reshape(64, 1, 256))
        d2 = jnp.concatenate(drs, axis=1).reshape(NQ, 256)
        t = lax.dot_general(d2, simil, (((1,), (1,)), ((), ())),
                            preferred_element_type=jnp.float32)
        cols = [jnp.sum(jnp.where(lane_iota == (q16 + p * 64), t, 0.0),
                        axis=1, keepdims=True) for p in range(4)]
        slot_cols.append(jnp.concatenate(cols, axis=1))
    ak = ak_ref[...]
    ki2 = lax.dot_general(ak, awx_ref[0], (((1,), (1,)), ((), ())),
                          preferred_element_type=jnp.float32)
    attn_add = lax.dot_general(q2, ki2, (((1,), (1,)), ((), ())),
                               preferred_element_type=jnp.float32)
    logits = jnp.concatenate(slot_cols + [attn_add], axis=1)
    m = jnp.max(logits, axis=1, keepdims=True)
    e = jnp.exp(logits - m)
    a = e / jnp.sum(e, axis=1, keepdims=True)
    wsum = jnp.zeros((NQ, 256), jnp.float32)
    for lvl in range(L):
        for p in range(4):
            wsum = wsum + a[:, lvl * 4 + p:lvl * 4 + p + 1] * \
                g_ref[lvl, 1024 * p:1024 * (p + 1), :]
    a16 = jnp.sum(a[:, :16], axis=1, keepdims=True)
    v_main = lax.dot_general(wsum, vw0_ref[0], (((1,), (1,)), ((), ())),
                             preferred_element_type=jnp.float32) + a16 * vb0_ref[0]
    v2 = lax.dot_general(ak, vw1_ref[0], (((1,), (1,)), ((), ())),
                         preferred_element_type=jnp.float32) + vb1_ref[0]
    v_add = lax.dot_general(a[:, 16:], v2, (((1,), (0,)), ((), ())),
                            preferred_element_type=jnp.float32)
    hm = hm_ref[...]
    hme = jnp.exp(hm - jnp.max(hm, axis=0, keepdims=True))
    hw = hme / jnp.sum(hme, axis=0, keepdims=True)
    hsel = lax.broadcasted_iota(jnp.int32, (9, 1), 0) == h
    hwh = jnp.sum(jnp.where(hsel, hw, 0.0), axis=0, keepdims=True)
    contrib = (v_main + v_add) * hwh

    @pl.when(h == 0)
    def _():
        out_ref[...] = q2 * hw[8:9, :] + contrib

    @pl.when(h != 0)
    def _():
        out_ref[...] = out_ref[...] + contrib


def _compute_idx(q2, rp_t, w, b2d):
    return pl.pallas_call(
        _idx_body,
        out_shape=jax.ShapeDtypeStruct((32, 4, NQ), jnp.int32),
    )(q2, rp_t, w, b2d)


@functools.lru_cache(maxsize=1)
def _sc_gather_fn():
    return functools.partial(
        pl.kernel,
        mesh=plsc.VectorSubcoreMesh(core_axis_name="c", subcore_axis_name="s"),
        out_type=jax.ShapeDtypeStruct((NROWS, D), jnp.float32),
        scratch_types=[
            pltpu.VMEM((SC_NCH, SC_CH), jnp.int32),
            pltpu.VMEM((SC_CH, D), jnp.float32),
            pltpu.VMEM((SC_CH, D), jnp.float32),
            pltpu.SemaphoreType.DMA,
            pltpu.SemaphoreType.DMA,
            pltpu.SemaphoreType.DMA,
            pltpu.SemaphoreType.DMA,
        ],
    )(_sc_gather_body)


def _head_stage(g, q2, qp, aw_main, aw_extra, vw0, vw1, vb0, vb1, ak, hm_t):
    return pl.pallas_call(
        _head_body,
        grid=(H,),
        in_specs=[
            pl.BlockSpec((4, 4096, 256), lambda h: (h, 0, 0)),
            pl.BlockSpec((NQ, 256), lambda h: (0, 0)),
            pl.BlockSpec((16, 64, 256), lambda h: (0, 0, 0)),
            pl.BlockSpec((4, 256, 256), lambda h: (h, 0, 0)),
            pl.BlockSpec((1, 256, 256), lambda h: (h, 0, 0)),
            pl.BlockSpec((1, 256, 256), lambda h: (h, 0, 0)),
            pl.BlockSpec((1, 256, 256), lambda h: (h, 0, 0)),
            pl.BlockSpec((1, 1, 256), lambda h: (h, 0, 0)),
            pl.BlockSpec((1, 1, 256), lambda h: (h, 0, 0)),
            pl.BlockSpec((64, 256), lambda h: (0, 0)),
            pl.BlockSpec((9, 256), lambda h: (0, 0)),
        ],
        out_specs=pl.BlockSpec((NQ, 256), lambda h: (0, 0)),
        out_shape=jax.ShapeDtypeStruct((NQ, 256), jnp.float32),
    )(g, q2, qp, aw_main, aw_extra, vw0, vw1, vb0, vb1, ak, hm_t)


def kernel(query, reference_points, input_flatten, input_spatial_shapes,
           add_keys, input_level_start_index, sampling_offsets_W,
           sampling_offsets_b, attn_W, value_W, value_b, head_mixer_W):
    q2 = query[0]
    rp_t = reference_points[0].transpose(1, 2, 0)
    b2d = sampling_offsets_b.reshape(256, 1)
    idx = _compute_idx(q2, rp_t, sampling_offsets_W, b2d)
    idx3 = idx.reshape(32, SC_NCH, SC_CH)
    g = _sc_gather_fn()(input_flatten[0], idx3)
    g3 = g.reshape(32, 4096, 256)
    qp = q2.reshape(64, 16, 256).transpose(1, 0, 2)
    out = _head_stage(
        g3, q2, qp,
        attn_W[:32], attn_W[4::4][:8],
        value_W[0::2], value_W[1::2],
        value_b[0::2].reshape(8, 1, 256), value_b[1::2].reshape(8, 1, 256),
        add_keys[0], head_mixer_W.T,
    )
    return out[None]


# speedup vs baseline: 7.5456x; 1.0078x over previous
import functools

import jax
import jax.numpy as jnp
from jax import lax
from jax.experimental import pallas as pl
from jax.experimental.pallas import tpu as pltpu
from jax.experimental.pallas import tpu_sc as plsc

H, L, P, NQ, D = 8, 4, 4, 1024, 256
SLEV = (128.0, 64.0, 32.0, 16.0)
START = (0, 16384, 20480, 21504)
NROWS = 32 * 4096
SC_CH = 128
SC_NCH = 32


def _idx_body(q2_ref, rp_ref, w_ref, b_ref, out_ref):
    ot = lax.dot_general(w_ref[...], q2_ref[...], (((1,), (1,)), ((), ())),
                         preferred_element_type=jnp.float32) + b_ref[...]
    ot3 = ot.reshape(128, 2, NQ)
    for g in range(32):
        lvl = g % 4
        s = SLEV[lvl]
        xg = ot3[4 * g:4 * g + 4, 0, :]
        yg = ot3[4 * g:4 * g + 4, 1, :]
        lx = jnp.clip(rp_ref[lvl, 0:1, :] + xg * (1.0 / s), 0.0, 0.999)
        ly = jnp.clip(rp_ref[lvl, 1:2, :] + yg * (1.0 / s), 0.0, 0.999)
        ix = (lx * s).astype(jnp.int32)
        iy = (ly * s).astype(jnp.int32)
        out_ref[g] = ix + iy * int(s) + START[lvl]


@functools.lru_cache(maxsize=None)
def _make_sc_gather(nrows):
    nch = nrows // (32 * SC_CH)
    rpw = nch * SC_CH

    def body(table_hbm, idx_hbm, out_hbm, idx_v, buf0, buf1,
             si0, si1, so0, so1):
        w = lax.axis_index("s") * 2 + lax.axis_index("c")
        pltpu.sync_copy(idx_hbm.at[w], idx_v)
        base = w * rpw
        pltpu.async_copy(table_hbm.at[idx_v.at[0]], buf0, si0)
        pltpu.async_copy(table_hbm.at[idx_v.at[1]], buf1, si1)

        def pair(t, carry):
            c0 = 2 * t
            pltpu.make_async_copy(table_hbm.at[idx_v.at[c0]], buf0, si0).wait()
            pltpu.async_copy(
                buf0, out_hbm.at[pl.ds(base + c0 * SC_CH, SC_CH)], so0)
            pltpu.make_async_copy(
                table_hbm.at[idx_v.at[c0 + 1]], buf1, si1).wait()
            pltpu.async_copy(
                buf1, out_hbm.at[pl.ds(base + (c0 + 1) * SC_CH, SC_CH)], so1)

            @pl.when(t < nch // 2 - 1)
            def _():
                pltpu.make_async_copy(
                    buf0, out_hbm.at[pl.ds(base + c0 * SC_CH, SC_CH)],
                    so0).wait()
                pltpu.async_copy(table_hbm.at[idx_v.at[c0 + 2]], buf0, si0)
                pltpu.make_async_copy(
                    buf1, out_hbm.at[pl.ds(base + (c0 + 1) * SC_CH, SC_CH)],
                    so1).wait()
                pltpu.async_copy(table_hbm.at[idx_v.at[c0 + 3]], buf1, si1)

            return carry

        lax.fori_loop(0, nch // 2, pair, 0)
        last = nch - 2
        pltpu.make_async_copy(
            buf0, out_hbm.at[pl.ds(base + last * SC_CH, SC_CH)], so0).wait()
        pltpu.make_async_copy(
            buf1, out_hbm.at[pl.ds(base + (last + 1) * SC_CH, SC_CH)],
            so1).wait()

    return functools.partial(
        pl.kernel,
        mesh=plsc.VectorSubcoreMesh(core_axis_name="c", subcore_axis_name="s"),
        out_type=jax.ShapeDtypeStruct((nrows, D), jnp.float32),
        scratch_types=[
            pltpu.VMEM((nch, SC_CH), jnp.int32),
            pltpu.VMEM((SC_CH, D), jnp.float32),
            pltpu.VMEM((SC_CH, D), jnp.float32),
            pltpu.SemaphoreType.DMA,
            pltpu.SemaphoreType.DMA,
            pltpu.SemaphoreType.DMA,
            pltpu.SemaphoreType.DMA,
        ],
    )(body)


def _head_body(hoff, g_ref, acc_ref, q2_ref, qp_ref, aw_ref, awx_ref,
               vw0_ref, vw1_ref, vb0_ref, vb1_ref, ak_ref, hm_ref, out_ref):
    h = pl.program_id(0)
    q2 = q2_ref[...]
    lane_iota = lax.broadcasted_iota(jnp.int32, (NQ, 256), 1)
    q16 = lax.broadcasted_iota(jnp.int32, (NQ, 256), 0) // 16
    slot_cols = []
    for lvl in range(L):
        simil = aw_ref[lvl]
        drs = []
        for r in range(16):
            kr = g_ref[lvl, 256 * r:256 * (r + 1), :]
            dr = lax.dot_general(qp_ref[r], kr, (((1,), (0,)), ((), ())),
                                 preferred_element_type=jnp.float32)
            drs.append(dr.reshape(64, 1, 256))
        d2 = jnp.concatenate(drs, axis=1).reshape(NQ, 256)
        t = lax.dot_general(d2, simil, (((1,), (1,)), ((), ())),
                            preferred_element_type=jnp.float32)
        cols = [jnp.sum(jnp.where(lane_iota == (q16 + p * 64), t, 0.0),
                        axis=1, keepdims=True) for p in range(4)]
        slot_cols.append(jnp.concatenate(cols, axis=1))
    ak = ak_ref[...]
    ki2 = lax.dot_general(ak, awx_ref[0], (((1,), (1,)), ((), ())),
                          preferred_element_type=jnp.float32)
    attn_add = lax.dot_general(q2, ki2, (((1,), (1,)), ((), ())),
                               preferred_element_type=jnp.float32)
    logits = jnp.concatenate(slot_cols + [attn_add], axis=1)
    m = jnp.max(logits, axis=1, keepdims=True)
    e = jnp.exp(logits - m)
    a = e / jnp.sum(e, axis=1, keepdims=True)
    wsum = jnp.zeros((NQ, 256), jnp.float32)
    for lvl in range(L):
        for p in range(4):
            wsum = wsum + a[:, lvl * 4 + p:lvl * 4 + p + 1] * \
                g_ref[lvl, 1024 * p:1024 * (p + 1), :]
    a16 = jnp.sum(a[:, :16], axis=1, keepdims=True)
    v_main = lax.dot_general(wsum, vw0_ref[0], (((1,), (1,)), ((), ())),
                             preferred_element_type=jnp.float32) + a16 * vb0_ref[0]
    v2 = lax.dot_general(ak, vw1_ref[0], (((1,), (1,)), ((), ())),
                         preferred_element_type=jnp.float32) + vb1_ref[0]
    v_add = lax.dot_general(a[:, 16:], v2, (((1,), (0,)), ((), ())),
                            preferred_element_type=jnp.float32)
    hm = hm_ref[...]
    hme = jnp.exp(hm - jnp.max(hm, axis=0, keepdims=True))
    hw = hme / jnp.sum(hme, axis=0, keepdims=True)
    hsel = lax.broadcasted_iota(jnp.int32, (9, 1), 0) == h + hoff
    hwh = jnp.sum(jnp.where(hsel, hw, 0.0), axis=0, keepdims=True)
    contrib = (v_main + v_add) * hwh

    @pl.when(h == 0)
    def _():
        base = acc_ref[...] + contrib
        if hoff == 0:
            base = base + q2 * hw[8:9, :]
        out_ref[...] = base

    @pl.when(h != 0)
    def _():
        out_ref[...] = out_ref[...] + contrib


def _compute_idx(q2, rp_t, w, b2d):
    return pl.pallas_call(
        _idx_body,
        out_shape=jax.ShapeDtypeStruct((32, 4, NQ), jnp.int32),
    )(q2, rp_t, w, b2d)


def _head_stage(hoff, nh, g, acc, q2, qp, attn_w, value_w, vb3, ak, hm_t):
    return pl.pallas_call(
        functools.partial(_head_body, hoff),
        grid=(nh,),
        in_specs=[
            pl.BlockSpec((4, 4096, 256), lambda h: (h, 0, 0)),
            pl.BlockSpec((NQ, 256), lambda h: (0, 0)),
            pl.BlockSpec((NQ, 256), lambda h: (0, 0)),
            pl.BlockSpec((16, 64, 256), lambda h: (0, 0, 0)),
            pl.BlockSpec((4, 256, 256), lambda h: (h + hoff, 0, 0)),
            pl.BlockSpec((1, 256, 256), lambda h: (4 * (h + hoff) + 4, 0, 0)),
            pl.BlockSpec((1, 256, 256), lambda h: (2 * (h + hoff), 0, 0)),
            pl.BlockSpec((1, 256, 256), lambda h: (2 * (h + hoff) + 1, 0, 0)),
            pl.BlockSpec((1, 1, 256), lambda h: (2 * (h + hoff), 0, 0)),
            pl.BlockSpec((1, 1, 256), lambda h: (2 * (h + hoff) + 1, 0, 0)),
            pl.BlockSpec((64, 256), lambda h: (0, 0)),
            pl.BlockSpec((9, 256), lambda h: (0, 0)),
        ],
        out_specs=pl.BlockSpec((NQ, 256), lambda h: (0, 0)),
        out_shape=jax.ShapeDtypeStruct((NQ, 256), jnp.float32),
    )(g, acc, q2, qp, attn_w, attn_w, value_w, value_w, vb3, vb3, ak, hm_t)


def kernel(query, reference_points, input_flatten, input_spatial_shapes,
           add_keys, input_level_start_index, sampling_offsets_W,
           sampling_offsets_b, attn_W, value_W, value_b, head_mixer_W):
    q2 = query[0]
    rp_t = reference_points[0].transpose(1, 2, 0)
    b2d = sampling_offsets_b.reshape(256, 1)
    idx = _compute_idx(q2, rp_t, sampling_offsets_W, b2d)
    idxf = idx.reshape(NROWS)
    half = NROWS // 2
    gather = _make_sc_gather(half)
    table = input_flatten[0]
    g1 = gather(table, idxf[:half].reshape(32, 16, SC_CH))
    g2 = gather(table, idxf[half:].reshape(32, 16, SC_CH))
    qp = q2.reshape(64, 16, 256).transpose(1, 0, 2)
    vb3 = value_b.reshape(16, 1, 256)
    hm_t = head_mixer_W.T
    ak = add_keys[0]
    zero = jnp.zeros((NQ, 256), jnp.float32)
    o1 = _head_stage(0, 4, g1.reshape(16, 4096, 256), zero, q2, qp,
                     attn_W, value_W, vb3, ak, hm_t)
    out = _head_stage(4, 4, g2.reshape(16, 4096, 256), o1, q2, qp,
                      attn_W, value_W, vb3, ak, hm_t)
    return out[None]
